# Initial kernel scaffold; baseline (speedup 1.0000x reference)
import jax
import jax.numpy as jnp
from jax.experimental import pallas as pl


def kernel(point_cloud, W_sub, bn_w0, bn_b0, W_conv0, bn_w1, bn_b1, W_conv1, bn_w2, bn_b2, W_conv2):
    raise NotImplementedError("write your pallas kernel here")


# speedup vs baseline: 1.4057x; 1.4057x over previous
import functools
import jax
import jax.numpy as jnp
from jax import lax
from jax.experimental import pallas as pl
from jax.experimental.pallas import tpu as pltpu

S = 128
B = 2
M = 4
EPS = 1e-4


def _sh(x, t, axis):
    if t == 0:
        return x
    pad = [(0, 0)] * x.ndim
    pad[axis] = (0, t)
    xp = jnp.pad(x, pad)
    idx = [slice(None)] * x.ndim
    idx[axis] = slice(t, t + x.shape[axis])
    return xp[tuple(idx)]


def _tail_kernel(y_ref, m_ref, bn1w_ref, bn1b_ref, cw1_ref, bn2w_ref,
                 bn2b_ref, cw2_ref, l8_ref, r8_ref, l8m_ref, r8m_ref,
                 l2_ref, r2_ref, out_ref):
    n1 = jnp.maximum(m_ref[0].sum() + m_ref[1].sum(), 1.0)
    y = [[y_ref[b * M + c] for c in range(M)] for b in range(B)]
    z = [[None] * M for _ in range(B)]
    for c in range(M):
        s1 = sum(y[b][c].sum() for b in range(B))
        s2 = sum((y[b][c] * y[b][c]).sum() for b in range(B))
        mean = s1 / n1
        var = s2 / n1 - mean * mean
        a = bn1w_ref[c] * lax.rsqrt(var + EPS)
        cc = bn1b_ref[c] - mean * a
        for b in range(B):
            z[b][c] = jnp.maximum((y[b][c] * a + cc) * m_ref[b], 0.0)

    y8 = [[None] * M for _ in range(B)]
    m8 = [None] * B
    for b in range(B):
        mm = jnp.maximum(m_ref[b], _sh(m_ref[b], 1, 2))
        mm = jnp.maximum(mm, _sh(mm, 1, 1))
        mm = jnp.maximum(mm, _sh(mm, 1, 0))
        m8[b] = jnp.stack([
            jnp.dot(jnp.dot(l8m_ref[...], mm[4 * d]), r8m_ref[...],
                    preferred_element_type=jnp.float32) for d in range(8)])
        for co in range(M):
            acc = jnp.zeros((32, 32, 32), jnp.float32)
            for td in range(2):
                for th in range(2):
                    for tw in range(2):
                        for ci in range(M):
                            w = cw1_ref[td * 4 + th * 2 + tw, ci * M + co]
                            zs = _sh(_sh(_sh(z[b][ci], tw, 2), th, 1), td, 0)
                            acc = acc + zs * w
            planes = []
            for d in range(8):
                p = acc[4 * d] + acc[4 * d + 2]
                planes.append(jnp.dot(jnp.dot(l8_ref[...], p), r8_ref[...],
                                      preferred_element_type=jnp.float32))
            y8[b][co] = jnp.stack(planes) * 0.5

    n2 = jnp.maximum(sum(jnp.sum(m8[b]) for b in range(B)), 1.0)
    z2 = [[None] * M for _ in range(B)]
    for c in range(M):
        s1 = sum(y8[b][c].sum() for b in range(B))
        s2 = sum((y8[b][c] * y8[b][c]).sum() for b in range(B))
        mean = s1 / n2
        var = s2 / n2 - mean * mean
        a = bn2w_ref[c] * lax.rsqrt(var + EPS)
        cc = bn2b_ref[c] - mean * a
        for b in range(B):
            z2[b][c] = jnp.maximum((y8[b][c] * a + cc) * m8[b], 0.0)

    for b in range(B):
        for co in range(M):
            acc = jnp.zeros((8, 8, 8), jnp.float32)
            for td in range(2):
                for th in range(2):
                    for tw in range(2):
                        for ci in range(M):
                            w = cw2_ref[td * 4 + th * 2 + tw, ci * M + co]
                            zs = _sh(_sh(_sh(z2[b][ci], tw, 2), th, 1), td, 0)
                            acc = acc + zs * w
            planes = []
            for d in range(2):
                p = acc[4 * d] + acc[4 * d + 2]
                planes.append(jnp.dot(jnp.dot(l2_ref[...], p), r2_ref[...],
                                      preferred_element_type=jnp.float32))
            out_ref[b * M + co] = jnp.stack(planes) * 0.5


def _tail(y32, m32, bn_w1, bn_b1, W_conv1, bn_w2, bn_b2, W_conv2):
    idx8 = jnp.arange(32)
    l8 = ((idx8[None, :] == 4 * jnp.arange(8)[:, None]) |
          (idx8[None, :] == 4 * jnp.arange(8)[:, None] + 2)).astype(jnp.float32) * 0.5
    r8 = l8.T
    l8m = (idx8[None, :] == 4 * jnp.arange(8)[:, None]).astype(jnp.float32)
    r8m = l8m.T
    idx2 = jnp.arange(8)
    l2 = ((idx2[None, :] == 4 * jnp.arange(2)[:, None]) |
          (idx2[None, :] == 4 * jnp.arange(2)[:, None] + 2)).astype(jnp.float32) * 0.5
    r2 = l2.T
    cw1 = W_conv1.reshape(8, 16)
    cw2 = W_conv2.reshape(8, 16)
    smem = pl.BlockSpec(memory_space=pltpu.SMEM)
    out = pl.pallas_call(
        _tail_kernel,
        out_shape=jax.ShapeDtypeStruct((B * M, 2, 2, 2), jnp.float32),
        in_specs=[pl.BlockSpec(memory_space=pltpu.VMEM)] * 2 +
                 [smem, smem, smem, smem, smem, smem] +
                 [pl.BlockSpec(memory_space=pltpu.VMEM)] * 6,
        out_specs=pl.BlockSpec(memory_space=pltpu.VMEM),
    )(y32.reshape(B * M, 32, 32, 32), m32, bn_w1, bn_b1, cw1,
      bn_w2, bn_b2, cw2, l8, r8, l8m, r8m, l2, r2)
    return out.reshape(B, M * 8)


def kernel(point_cloud, W_sub, bn_w0, bn_b0, W_conv0, bn_w1, bn_b1, W_conv1,
           bn_w2, bn_b2, W_conv2):
    pc = point_cloud
    ix = pc[:, 0].astype(jnp.int32)
    iy = pc[:, 1].astype(jnp.int32)
    iz = pc[:, 2].astype(jnp.int32)
    ib = pc[:, 3].astype(jnp.int32)
    lin = ((ib * S + ix) * S + iy) * S + iz
    dense = jnp.zeros((B * S * S * S,), jnp.float32).at[lin].add(pc[:, 4])
    cnt = jnp.zeros((B * S * S * S,), jnp.float32).at[lin].add(1.0)
    dense = dense.reshape(B, 1, S, S, S)
    mask = (cnt > 0).astype(jnp.float32).reshape(B, 1, S, S, S)

    x = lax.conv_general_dilated(
        dense, W_sub, (1, 1, 1), 'SAME',
        dimension_numbers=('NCDHW', 'DHWIO', 'NCDHW')) * mask
    n0 = jnp.maximum(jnp.sum(mask), 1.0)
    s1 = jnp.sum(x, axis=(0, 2, 3, 4))
    s2 = jnp.sum(x * x, axis=(0, 2, 3, 4))
    mean = s1 / n0
    var = s2 / n0 - mean * mean
    a = bn_w0 / jnp.sqrt(var + EPS)
    c = bn_b0 - mean * a
    z = jnp.maximum((x * a[None, :, None, None, None]
                     + c[None, :, None, None, None]) * mask, 0.0)
    y64 = lax.conv_general_dilated(
        z, W_conv0, (2, 2, 2), 'VALID',
        dimension_numbers=('NCDHW', 'DHWIO', 'NCDHW'))
    y32 = lax.reduce_window(y64, 0.0, lax.add, (1, 1, 2, 2, 2),
                            (1, 1, 2, 2, 2), 'VALID') / 8.0
    m32 = lax.reduce_window(mask, -jnp.inf, lax.max, (1, 1, 4, 4, 4),
                            (1, 1, 4, 4, 4), 'VALID')
    return _tail(y32, m32[:, 0], bn_w1, bn_b1, W_conv1, bn_w2, bn_b2, W_conv2)
